# Initial kernel scaffold; baseline (speedup 1.0000x reference)
import jax
import jax.numpy as jnp
from jax.experimental import pallas as pl


def kernel(indices, table):
    raise NotImplementedError("write your pallas kernel here")


# speedup vs baseline: 13.6034x; 13.6034x over previous
import functools

import jax
import jax.numpy as jnp
from jax import lax
from jax.experimental import pallas as pl
from jax.experimental.pallas import tpu as pltpu
from jax.experimental.pallas import tpu_sc as plsc

CHUNK = 128
NBUF = 8


def _gather_call(idx3, table, n_workers, n_chunks, embed_dim):
    num_cores = plsc.get_sparse_core_info().num_cores
    b_per_w = n_chunks * CHUNK
    total = n_workers * b_per_w
    mesh = plsc.VectorSubcoreMesh(core_axis_name="c", subcore_axis_name="s")

    @functools.partial(
        pl.kernel,
        mesh=mesh,
        out_type=jax.ShapeDtypeStruct((total, embed_dim), jnp.float32),
        compiler_params=pltpu.CompilerParams(use_tc_tiling_on_sc=False),
        scratch_types=[
            pltpu.VMEM((n_chunks, CHUNK), jnp.int32),
            pltpu.VMEM((NBUF, CHUNK, embed_dim), jnp.float32),
            pltpu.SemaphoreType.DMA,
        ],
    )
    def body(idx_hbm, table_hbm, out_hbm, idx_v, rows_v, sem):
        wid = lax.axis_index("s") * num_cores + lax.axis_index("c")
        base = wid * b_per_w
        pltpu.sync_copy(idx_hbm.at[wid], idx_v)

        for b in range(NBUF):
            pltpu.async_copy(table_hbm.at[idx_v.at[b]], rows_v.at[b], sem)

        n_groups = n_chunks // NBUF

        def group(g, carry):
            for b in range(NBUF):
                j = g * NBUF + b
                pltpu.make_async_copy(
                    table_hbm.at[idx_v.at[j]], rows_v.at[b], sem
                ).wait()
                pltpu.sync_copy(
                    rows_v.at[b], out_hbm.at[pl.ds(base + j * CHUNK, CHUNK)]
                )
                pltpu.async_copy(
                    table_hbm.at[idx_v.at[j + NBUF]], rows_v.at[b], sem
                )
            return carry

        lax.fori_loop(0, n_groups - 1, group, 0)

        for b in range(NBUF):
            j = (n_groups - 1) * NBUF + b
            pltpu.make_async_copy(
                table_hbm.at[idx_v.at[j]], rows_v.at[b], sem
            ).wait()
            pltpu.sync_copy(
                rows_v.at[b], out_hbm.at[pl.ds(base + j * CHUNK, CHUNK)]
            )

    return body(idx3, table)


def kernel(indices, table):
    original_shape = indices.shape
    embed_dim = table.shape[1]
    flat = indices.reshape(-1).astype(jnp.int32)
    info = plsc.get_sparse_core_info()
    n_workers = info.num_cores * info.num_subcores
    n_chunks = flat.size // (n_workers * CHUNK)
    idx3 = flat.reshape(n_workers, n_chunks, CHUNK)
    out = _gather_call(idx3, table, n_workers, n_chunks, embed_dim)
    return out.reshape(original_shape + (embed_dim,))
